# baseline (speedup 1.0000x reference)
import functools

import jax
import jax.numpy as jnp
from jax import lax
from jax.experimental import pallas as pl
from jax.experimental.pallas import tpu as pltpu
from jax.experimental.pallas import tpu_sc as plsc

N = 10000
NPAD = 10240
E = 320000
NIMG = 100
NPAIR = 1024
NC, NS = 2, 16
NW = NC * NS
CH = 128
NCHUNK = (NPAD * NW) // (NW * CH)
EPAD = NW * 10240
NCHUNK = 10240 // CH
RPT = NPAD // NS
NB = 4

def _mesh():
  return plsc.VectorSubcoreMesh(
      core_axis_name="c", subcore_axis_name="s", num_cores=NC, num_subcores=NS)


@functools.cache
def _get_sc_degree():
  @functools.partial(
      pl.kernel,
      out_type=jax.ShapeDtypeStruct((NW, NPAD), jnp.float32),
      mesh=_mesh(),
      scratch_types=[
          pltpu.VMEM((NCHUNK, CH), jnp.int32),
          pltpu.VMEM((NPAD,), jnp.float32),
      ],
      compiler_params=pltpu.CompilerParams(
          needs_layout_passes=False, use_tc_tiling_on_sc=False),
  )
  def _sc_degree(dst_hbm, out_hbm, idx_v, acc_v):
    c = lax.axis_index("c")
    s = lax.axis_index("s")
    w = c * NS + s
    pltpu.sync_copy(dst_hbm.at[w], idx_v)
    zeros16 = jnp.zeros((16,), jnp.float32)

    def zero_body(i, carry):
      acc_v[pl.ds(i * 16, 16)] = zeros16
      return carry

    lax.fori_loop(0, NPAD // 16, zero_body, 0)
    ones16 = jnp.ones((16,), jnp.float32)

    def chunk_body(j, carry):
      def vec_body(i, carry2):
        idx = idx_v[j, pl.ds(i * 16, 16)]
        plsc.addupdate_scatter(acc_v, [idx], ones16)
        return carry2

      return lax.fori_loop(0, CH // 16, vec_body, carry)

    lax.fori_loop(0, NCHUNK, chunk_body, 0)
    pltpu.sync_copy(acc_v, out_hbm.at[w])

  return _sc_degree


@functools.cache
def _get_msgpass(D):
  NG = NCHUNK // NB

  @functools.partial(
      pl.kernel,
      out_type=jax.ShapeDtypeStruct((NC, NPAD, D), jnp.float32),
      mesh=_mesh(),
      scratch_types=[
          pltpu.VMEM((NCHUNK, CH), jnp.int32),
          pltpu.VMEM((NCHUNK, CH), jnp.int32),
          pltpu.VMEM((2 * NB, CH, D), jnp.float32),
          pltpu.VMEM_SHARED((NPAD, D), jnp.float32),
          pltpu.SemaphoreType.DMA,
          pltpu.SemaphoreType.DMA,
          pltpu.SemaphoreType.DMA,
      ],
      compiler_params=pltpu.CompilerParams(
          needs_layout_passes=False, use_tc_tiling_on_sc=False),
  )
  def msgpass(hs_hbm, src_hbm, dst_hbm, out_hbm, src_v, dst_v, rows_v,
              acc_sh, gsem0, gsem1, ssem):
    c = lax.axis_index("c")
    s = lax.axis_index("s")
    w = c * NS + s
    gsems = (gsem0, gsem1)
    pltpu.make_async_copy(src_hbm.at[w], src_v, gsem0).start()
    pltpu.make_async_copy(dst_hbm.at[w], dst_v, gsem1).start()

    zeros16 = jnp.zeros((16,), jnp.float32)

    def zero_body(r, carry):
      for q in range(D // 16):
        rows_v[0, r, pl.ds(q * 16, 16)] = zeros16
      return carry

    lax.fori_loop(0, CH, zero_body, 0)
    pltpu.make_async_copy(src_hbm.at[w], src_v, gsem0).wait()
    pltpu.make_async_copy(dst_hbm.at[w], dst_v, gsem1).wait()
    for r in range(RPT // CH):
      pltpu.sync_copy(rows_v.at[0], acc_sh.at[pl.ds(s * RPT + r * CH, CH)])

    def gather(j, slot, sem):
      return pltpu.make_async_copy(hs_hbm.at[src_v.at[j]], rows_v.at[slot],
                                   sem)

    def scatter_start(j, slot):
      pltpu.async_copy(rows_v.at[slot], acc_sh.at[dst_v.at[j]], ssem,
                       add=True)

    def scatter_wait(j, slot):
      pltpu.make_async_copy(rows_v.at[slot], acc_sh.at[dst_v.at[j]],
                            ssem).wait()

    for b in range(NB):
      gather(b, b, gsem0).start()
    plsc.subcore_barrier()

    def pair_body(i, carry):
      for bank in range(2):
        g = 2 * i + bank
        other = 1 - bank
        nxt = g + 1

        @pl.when(jnp.logical_and(nxt < NG, g >= 1))
        def _():
          for b in range(NB):
            scatter_wait((g - 1) * NB + b, other * NB + b)

        @pl.when(nxt < NG)
        def _():
          for b in range(NB):
            gather(nxt * NB + b, other * NB + b, gsems[other]).start()

        for b in range(NB):
          gather(g * NB + b, bank * NB + b, gsems[bank]).wait()
        for b in range(NB):
          scatter_start(g * NB + b, bank * NB + b)
      return carry

    lax.fori_loop(0, NG // 2, pair_body, 0)
    for b in range(NB):
      scatter_wait((NG - 2) * NB + b, 0 * NB + b)
    for b in range(NB):
      scatter_wait((NG - 1) * NB + b, 1 * NB + b)
    plsc.subcore_barrier()
    pltpu.sync_copy(acc_sh.at[pl.ds(s * RPT, RPT)],
                    out_hbm.at[c, pl.ds(s * RPT, RPT)])

  return msgpass


def _dinv_col(degp_blk):
  ones = jnp.ones((NW, 1), jnp.float32)
  deg = lax.dot_general(degp_blk, ones, (((0,), (0,)), ((), ())),
                        preferred_element_type=jnp.float32)
  return lax.rsqrt(1.0 + deg)


def _tc1_body(x_ref, w1_ref, degp_ref, hs1a_ref, hs1b_ref):
  h0 = jnp.dot(x_ref[...], w1_ref[...], preferred_element_type=jnp.float32)
  hs1 = h0 * _dinv_col(degp_ref[...])
  hs1a_ref[...] = hs1[:, :64]
  hs1b_ref[...] = hs1[:, 64:]


def _tc2_body(pa_ref, pb_ref, hs1a_ref, hs1b_ref, degp_ref, b1_ref, w2_ref,
              hs2_ref):
  dinv = _dinv_col(degp_ref[...])
  acc = jnp.concatenate(
      [pa_ref[0] + pa_ref[1] + hs1a_ref[...],
       pb_ref[0] + pb_ref[1] + hs1b_ref[...]], axis=1)
  h1 = jnp.maximum(acc * dinv + b1_ref[...], 0.0)
  hs2_ref[...] = jnp.dot(h1, w2_ref[...],
                         preferred_element_type=jnp.float32) * dinv


def _tc3_body(q_ref, hs2_ref, degp_ref, b2_ref, batch_ref, p0_ref, p1_ref,
              fca_ref, fcb_ref, fcb0_ref, out_ref):
  dinv = _dinv_col(degp_ref[...])
  h2 = (q_ref[0] + q_ref[1] + hs2_ref[...]) * dinv + b2_ref[...]
  ids = lax.broadcasted_iota(jnp.int32, (NIMG, NPAD), 0).astype(jnp.float32)
  oneh = (batch_ref[...] == ids).astype(jnp.float32)
  counts = jnp.dot(oneh, jnp.ones((NPAD, 1), jnp.float32),
                   preferred_element_type=jnp.float32)
  sums = jnp.dot(oneh, h2, preferred_element_type=jnp.float32)
  img = sums / jnp.maximum(counts, 1.0)
  sv = jnp.dot(img, fca_ref[...], preferred_element_type=jnp.float32)
  tv = jnp.dot(img, fcb_ref[...], preferred_element_type=jnp.float32)
  pid = lax.broadcasted_iota(jnp.int32, (NPAIR, NIMG), 1).astype(jnp.float32)
  oh0 = (p0_ref[...] == pid).astype(jnp.float32)
  oh1 = (p1_ref[...] == pid).astype(jnp.float32)
  z = (jnp.dot(oh0, sv, preferred_element_type=jnp.float32)
       + jnp.dot(oh1, tv, preferred_element_type=jnp.float32)
       + fcb0_ref[...])
  out_ref[...] = 1.0 / (1.0 + jnp.exp(-z))


_BLK = 1024
_GRID = NPAD // _BLK


def _tc1(xp, W1, degp):
  return pl.pallas_call(
      _tc1_body,
      grid=(_GRID,),
      in_specs=[
          pl.BlockSpec((_BLK, 128), lambda i: (i, 0)),
          pl.BlockSpec((128, 128), lambda i: (0, 0)),
          pl.BlockSpec((NW, _BLK), lambda i: (0, i)),
      ],
      out_specs=[pl.BlockSpec((_BLK, 64), lambda i: (i, 0)),
                 pl.BlockSpec((_BLK, 64), lambda i: (i, 0))],
      out_shape=[jax.ShapeDtypeStruct((NPAD, 64), jnp.float32),
                 jax.ShapeDtypeStruct((NPAD, 64), jnp.float32)],
  )(xp, W1, degp)


def _tc2(pa, pb, hs1a, hs1b, degp, b1, W2):
  return pl.pallas_call(
      _tc2_body,
      grid=(_GRID,),
      in_specs=[
          pl.BlockSpec((NC, _BLK, 64), lambda i: (0, i, 0)),
          pl.BlockSpec((NC, _BLK, 64), lambda i: (0, i, 0)),
          pl.BlockSpec((_BLK, 64), lambda i: (i, 0)),
          pl.BlockSpec((_BLK, 64), lambda i: (i, 0)),
          pl.BlockSpec((NW, _BLK), lambda i: (0, i)),
          pl.BlockSpec((1, 128), lambda i: (0, 0)),
          pl.BlockSpec((128, 64), lambda i: (0, 0)),
      ],
      out_specs=pl.BlockSpec((_BLK, 64), lambda i: (i, 0)),
      out_shape=jax.ShapeDtypeStruct((NPAD, 64), jnp.float32),
  )(pa, pb, hs1a, hs1b, degp, b1, W2)


def _tc3(q, hs2, degp, b2, batchf, p0f, p1f, fca, fcb, fcb0):
  return pl.pallas_call(
      _tc3_body,
      out_shape=jax.ShapeDtypeStruct((NPAIR, 1), jnp.float32),
  )(q, hs2, degp, b2, batchf, p0f, p1f, fca, fcb, fcb0)


def kernel(x, edge_index, batch, pairs, W1, b1, W2, b2, fc_W, fc_b):
  f32 = jnp.float32
  xp = jnp.concatenate([x, jnp.zeros((NPAD - N, 128), f32)], axis=0)
  pad = N + (jnp.arange(EPAD - E, dtype=jnp.int32) % (NPAD - N))
  src_r = jnp.concatenate([edge_index[0], pad]).reshape(NW, NCHUNK, CH)
  dst_r = jnp.concatenate([edge_index[1], pad]).reshape(NW, NCHUNK, CH)
  batchf = jnp.concatenate(
      [batch.astype(f32), jnp.full((NPAD - N,), 1e9, f32)]).reshape(1, NPAD)
  p0f = pairs[:, 0:1].astype(f32)
  p1f = pairs[:, 1:2].astype(f32)

  degp = _get_sc_degree()(dst_r)
  hs1a, hs1b = _tc1(xp, W1, degp)
  mp = _get_msgpass(64)
  pa = mp(hs1a, src_r, dst_r)
  pb = mp(hs1b, src_r, dst_r)
  hs2 = _tc2(pa, pb, hs1a, hs1b, degp, b1.reshape(1, 128), W2)
  q = mp(hs2, src_r, dst_r)
  return _tc3(q, hs2, degp, b2.reshape(1, 64), batchf, p0f, p1f,
              fc_W[:64], fc_W[64:], fc_b.reshape(1, 1))


# speedup vs baseline: 1.0502x; 1.0502x over previous
import functools

import jax
import jax.numpy as jnp
from jax import lax
from jax.experimental import pallas as pl
from jax.experimental.pallas import tpu as pltpu
from jax.experimental.pallas import tpu_sc as plsc

N = 10000
NPAD = 10240
E = 320000
NIMG = 100
NPAIR = 1024
NC, NS = 2, 16
NW = NC * NS
CH = 128
NCHUNK = (NPAD * NW) // (NW * CH)
EPAD = NW * 10240
NCHUNK = 10240 // CH
RPT = NPAD // NS

def _mesh():
  return plsc.VectorSubcoreMesh(
      core_axis_name="c", subcore_axis_name="s", num_cores=NC, num_subcores=NS)


@functools.cache
def _get_sc_degree():
  @functools.partial(
      pl.kernel,
      out_type=jax.ShapeDtypeStruct((NW, NPAD), jnp.float32),
      mesh=_mesh(),
      scratch_types=[
          pltpu.VMEM((NCHUNK, CH), jnp.int32),
          pltpu.VMEM((NPAD,), jnp.float32),
      ],
      compiler_params=pltpu.CompilerParams(
          needs_layout_passes=False, use_tc_tiling_on_sc=False),
  )
  def _sc_degree(dst_hbm, out_hbm, idx_v, acc_v):
    c = lax.axis_index("c")
    s = lax.axis_index("s")
    w = c * NS + s
    pltpu.sync_copy(dst_hbm.at[w], idx_v)
    zeros16 = jnp.zeros((16,), jnp.float32)

    def zero_body(i, carry):
      acc_v[pl.ds(i * 16, 16)] = zeros16
      return carry

    lax.fori_loop(0, NPAD // 16, zero_body, 0)
    ones16 = jnp.ones((16,), jnp.float32)

    def chunk_body(j, carry):
      def vec_body(i, carry2):
        idx = idx_v[j, pl.ds(i * 16, 16)]
        plsc.addupdate_scatter(acc_v, [idx], ones16)
        return carry2

      return lax.fori_loop(0, CH // 16, vec_body, carry)

    lax.fori_loop(0, NCHUNK, chunk_body, 0)
    pltpu.sync_copy(acc_v, out_hbm.at[w])

  return _sc_degree


@functools.cache
def _get_msgpass(D):
  nch = NCHUNK * 2
  NB = 2 if D == 64 else 4
  NG = nch // NB

  @functools.partial(
      pl.kernel,
      out_type=jax.ShapeDtypeStruct((NC, NPAD, D), jnp.float32),
      mesh=_mesh(),
      scratch_types=[
          pltpu.VMEM((nch, CH), jnp.int32),
          pltpu.VMEM((nch, CH), jnp.int32),
          pltpu.VMEM((2 * NB, CH, D), jnp.float32),
          pltpu.VMEM_SHARED((NPAD, D), jnp.float32),
          pltpu.SemaphoreType.DMA,
          pltpu.SemaphoreType.DMA,
          pltpu.SemaphoreType.DMA,
      ],
      compiler_params=pltpu.CompilerParams(
          needs_layout_passes=False, use_tc_tiling_on_sc=False),
  )
  def msgpass(hs_hbm, src_hbm, dst_hbm, out_hbm, src_v, dst_v, rows_v,
              acc_sh, gsem0, gsem1, ssem):
    c = lax.axis_index("c")
    s = lax.axis_index("s")
    gsems = (gsem0, gsem1)
    src_slab = src_hbm.at[c, s]
    dst_slab = dst_hbm.at[s]
    pltpu.make_async_copy(src_slab, src_v, gsem0).start()
    pltpu.make_async_copy(dst_slab, dst_v, gsem1).start()

    zeros16 = jnp.zeros((16,), jnp.float32)

    def zero_body(r, carry):
      for q in range(D // 16):
        rows_v[0, r, pl.ds(q * 16, 16)] = zeros16
      return carry

    lax.fori_loop(0, CH, zero_body, 0)
    pltpu.make_async_copy(src_slab, src_v, gsem0).wait()
    pltpu.make_async_copy(dst_slab, dst_v, gsem1).wait()
    for r in range(RPT // CH):
      pltpu.sync_copy(rows_v.at[0], acc_sh.at[pl.ds(s * RPT + r * CH, CH)])

    def gather(j, slot, sem):
      return pltpu.make_async_copy(hs_hbm.at[src_v.at[j]], rows_v.at[slot],
                                   sem)

    def scatter_start(j, slot):
      pltpu.async_copy(rows_v.at[slot], acc_sh.at[dst_v.at[j]], ssem,
                       add=True)

    def scatter_wait(j, slot):
      pltpu.make_async_copy(rows_v.at[slot], acc_sh.at[dst_v.at[j]],
                            ssem).wait()

    for b in range(NB):
      gather(b, b, gsem0).start()
    plsc.subcore_barrier()

    def pair_body(i, carry):
      for bank in range(2):
        g = 2 * i + bank
        other = 1 - bank
        nxt = g + 1

        @pl.when(jnp.logical_and(nxt < NG, g >= 1))
        def _():
          for b in range(NB):
            scatter_wait((g - 1) * NB + b, other * NB + b)

        @pl.when(nxt < NG)
        def _():
          for b in range(NB):
            gather(nxt * NB + b, other * NB + b, gsems[other]).start()

        for b in range(NB):
          gather(g * NB + b, bank * NB + b, gsems[bank]).wait()
        for b in range(NB):
          scatter_start(g * NB + b, bank * NB + b)
      return carry

    lax.fori_loop(0, NG // 2, pair_body, 0)
    for b in range(NB):
      scatter_wait((NG - 2) * NB + b, 0 * NB + b)
    for b in range(NB):
      scatter_wait((NG - 1) * NB + b, 1 * NB + b)
    plsc.subcore_barrier()
    pltpu.sync_copy(acc_sh.at[pl.ds(s * RPT, RPT)],
                    out_hbm.at[c, pl.ds(s * RPT, RPT)])

  return msgpass


def _dinv_col(degp_blk):
  ones = jnp.ones((NW, 1), jnp.float32)
  deg = lax.dot_general(degp_blk, ones, (((0,), (0,)), ((), ())),
                        preferred_element_type=jnp.float32)
  return lax.rsqrt(1.0 + deg)


def _tc1_body(x_ref, w1_ref, degp_ref, hst_ref):
  h0 = jnp.dot(x_ref[...], w1_ref[...], preferred_element_type=jnp.float32)
  hs1 = h0 * _dinv_col(degp_ref[...])
  hst_ref[0] = hs1[:, :64]
  hst_ref[1] = hs1[:, 64:]


def _tc2_body(p_ref, hst_ref, degp_ref, b1_ref, w2_ref, hs2_ref):
  dinv = _dinv_col(degp_ref[...])
  acc = jnp.concatenate(
      [p_ref[0] + hst_ref[0], p_ref[1] + hst_ref[1]], axis=1)
  h1 = jnp.maximum(acc * dinv + b1_ref[...], 0.0)
  hs2 = jnp.dot(h1, w2_ref[...], preferred_element_type=jnp.float32) * dinv
  hs2_ref[0] = hs2[:, :32]
  hs2_ref[1] = hs2[:, 32:]


def _tc3_body(q_ref, hs2_ref, degp_ref, b2_ref, batch_ref, p0_ref, p1_ref,
              fca_ref, fcb_ref, fcb0_ref, out_ref):
  dinv = _dinv_col(degp_ref[...])
  acc = jnp.concatenate(
      [q_ref[0] + hs2_ref[0], q_ref[1] + hs2_ref[1]], axis=1)
  h2 = acc * dinv + b2_ref[...]
  ids = lax.broadcasted_iota(jnp.int32, (NIMG, NPAD), 0).astype(jnp.float32)
  oneh = (batch_ref[...] == ids).astype(jnp.float32)
  counts = jnp.dot(oneh, jnp.ones((NPAD, 1), jnp.float32),
                   preferred_element_type=jnp.float32)
  sums = jnp.dot(oneh, h2, preferred_element_type=jnp.float32)
  img = sums / jnp.maximum(counts, 1.0)
  sv = jnp.dot(img, fca_ref[...], preferred_element_type=jnp.float32)
  tv = jnp.dot(img, fcb_ref[...], preferred_element_type=jnp.float32)
  pid = lax.broadcasted_iota(jnp.int32, (NPAIR, NIMG), 1).astype(jnp.float32)
  oh0 = (p0_ref[...] == pid).astype(jnp.float32)
  oh1 = (p1_ref[...] == pid).astype(jnp.float32)
  z = (jnp.dot(oh0, sv, preferred_element_type=jnp.float32)
       + jnp.dot(oh1, tv, preferred_element_type=jnp.float32)
       + fcb0_ref[...])
  out_ref[...] = 1.0 / (1.0 + jnp.exp(-z))


_BLK = 1024
_GRID = NPAD // _BLK


def _tc1(xp, W1, degp):
  return pl.pallas_call(
      _tc1_body,
      grid=(_GRID,),
      in_specs=[
          pl.BlockSpec((_BLK, 128), lambda i: (i, 0)),
          pl.BlockSpec((128, 128), lambda i: (0, 0)),
          pl.BlockSpec((NW, _BLK), lambda i: (0, i)),
      ],
      out_specs=pl.BlockSpec((2, _BLK, 64), lambda i: (0, i, 0)),
      out_shape=jax.ShapeDtypeStruct((2, NPAD, 64), jnp.float32),
  )(xp, W1, degp)


def _tc2(p, hst, degp, b1, W2):
  return pl.pallas_call(
      _tc2_body,
      grid=(_GRID,),
      in_specs=[
          pl.BlockSpec((2, _BLK, 64), lambda i: (0, i, 0)),
          pl.BlockSpec((2, _BLK, 64), lambda i: (0, i, 0)),
          pl.BlockSpec((NW, _BLK), lambda i: (0, i)),
          pl.BlockSpec((1, 128), lambda i: (0, 0)),
          pl.BlockSpec((128, 64), lambda i: (0, 0)),
      ],
      out_specs=pl.BlockSpec((2, _BLK, 32), lambda i: (0, i, 0)),
      out_shape=jax.ShapeDtypeStruct((2, NPAD, 32), jnp.float32),
  )(p, hst, degp, b1, W2)


def _tc3(q, hs2, degp, b2, batchf, p0f, p1f, fca, fcb, fcb0):
  return pl.pallas_call(
      _tc3_body,
      out_shape=jax.ShapeDtypeStruct((NPAIR, 1), jnp.float32),
  )(q, hs2, degp, b2, batchf, p0f, p1f, fca, fcb, fcb0)


def kernel(x, edge_index, batch, pairs, W1, b1, W2, b2, fc_W, fc_b):
  f32 = jnp.float32
  xp = jnp.concatenate([x, jnp.zeros((NPAD - N, 128), f32)], axis=0)
  pad = N + (jnp.arange(EPAD - E, dtype=jnp.int32) % (NPAD - N))
  src_all = jnp.concatenate([edge_index[0], pad])
  dst_all = jnp.concatenate([edge_index[1], pad])
  src_r = src_all.reshape(NW, NCHUNK, CH)
  dst_r = dst_all.reshape(NW, NCHUNK, CH)
  src_base = src_all.reshape(NS, 2 * NCHUNK, CH)
  src_d = jnp.stack([src_base, src_base + NPAD])
  dst_d = dst_all.reshape(NS, 2 * NCHUNK, CH)
  batchf = jnp.concatenate(
      [batch.astype(f32), jnp.full((NPAD - N,), 1e9, f32)]).reshape(1, NPAD)
  p0f = pairs[:, 0:1].astype(f32)
  p1f = pairs[:, 1:2].astype(f32)

  degp = _get_sc_degree()(dst_r)
  hst = _tc1(xp, W1, degp)
  p = _get_msgpass(64)(hst.reshape(2 * NPAD, 64), src_d, dst_d)
  hst2 = _tc2(p, hst, degp, b1.reshape(1, 128), W2)
  q = _get_msgpass(32)(hst2.reshape(2 * NPAD, 32), src_d, dst_d)
  return _tc3(q, hst2, degp, b2.reshape(1, 64), batchf, p0f, p1f,
              fc_W[:64], fc_W[64:], fc_b.reshape(1, 1))
